# baseline (speedup 1.0000x reference)
import jax
import jax.numpy as jnp
from jax import lax
from jax.experimental import pallas as pl
from jax.experimental.pallas import tpu as pltpu
from jax.experimental.pallas import tpu_sc as plsc

N = 10000
D = 128
HD = 64
E = 320000

NPAD = 10240
NT = 16
EW = 20480
EPAD = NT * EW
CH = 80
NCH = EW // CH
K = 4
NG = NCH // K
NACC = NPAD
SA = NACC // NT
RB = 256

_SC_PARAMS = pltpu.CompilerParams(use_tc_tiling_on_sc=False)
_MESH = plsc.VectorSubcoreMesh(core_axis_name="c", subcore_axis_name="s")


W = 16
NWIN = NCH // W


def _agg_body(h, src, dst, agg_o, sw0, dw0, sw1, dw1, *rest):
  bufs = rest[:2 * K]
  hs, acc, sg, ss, si = rest[2 * K:]
  seta = bufs[:K]
  setb = bufs[K:]
  wins = ((sw0, dw0), (sw1, dw1))
  c = lax.axis_index("c")
  s = lax.axis_index("s")
  r0 = bufs[0]

  pltpu.async_copy(h.at[c, pl.ds(s * SA, SA), :], hs.at[pl.ds(s * SA, SA), :],
                   si)
  pltpu.async_copy(src.at[s, pl.ds(0, W), :], sw0, si)
  pltpu.async_copy(dst.at[s, pl.ds(0, W), :], dw0, si)

  z16 = jnp.zeros((16,), jnp.float32)

  def zero_row(i, _):
    for j in range(HD // 16):
      r0[i, pl.ds(j * 16, 16)] = z16
    return 0

  lax.fori_loop(0, CH, zero_row, 0)
  for k in range(SA // CH):
    pltpu.sync_copy(r0, acc.at[pl.ds(s * SA + k * CH, CH), :])

  pltpu.make_async_copy(h.at[c, pl.ds(s * SA, SA), :],
                        hs.at[pl.ds(s * SA, SA), :], si).wait()
  pltpu.make_async_copy(src.at[s, pl.ds(0, W), :], sw0, si).wait()
  pltpu.make_async_copy(dst.at[s, pl.ds(0, W), :], dw0, si).wait()

  plsc.subcore_barrier()

  def issue_gathers(q, win, bset):
    for j in range(K):
      pltpu.async_copy(hs.at[win[0].at[K * q + j]], bset[j], sg)

  def drain_gathers(bset):
    for j in range(K):
      pltpu.make_async_copy(hs.at[sw0.at[0]], bset[j], sg).wait()

  def issue_scatters(q, win, bset):
    for j in range(K):
      pltpu.async_copy(bset[j], acc.at[win[1].at[K * q + j]], ss, add=True)

  def drain_scatters(bset):
    for j in range(K):
      pltpu.make_async_copy(bset[j], acc.at[dw0.at[0]], ss).wait()

  def prefetch_into(w, pair):
    pltpu.async_copy(src.at[s, pl.ds(w * W, W), :], wins[pair][0], si)
    pltpu.async_copy(dst.at[s, pl.ds(w * W, W), :], wins[pair][1], si)

  def drain_win():
    pltpu.make_async_copy(src.at[s, pl.ds(0, W), :], sw0, si).wait()
    pltpu.make_async_copy(dst.at[s, pl.ds(0, W), :], dw0, si).wait()

  def gstep(q, wq, qn, wqn, cur, nxt):
    drain_scatters(nxt)
    issue_gathers(qn, wqn, nxt)
    drain_gathers(cur)
    issue_scatters(q, wq, cur)

  issue_gathers(0, wins[0], seta)
  issue_gathers(1, wins[0], setb)
  drain_gathers(seta)
  issue_scatters(0, wins[0], seta)

  def window_body(w, pair):
    wc = wins[pair]
    wn = wins[1 - pair]
    gstep(1, wc, 2, wc, setb, seta)
    prefetch_into(w + 1, 1 - pair)
    gstep(2, wc, 3, wc, seta, setb)
    drain_win()
    gstep(3, wc, 0, wn, setb, seta)
    gstep(0, wn, 1, wn, seta, setb)

  def loop_body(m, _):
    window_body(2 * m, 0)
    window_body(2 * m + 1, 1)
    return 0

  lax.fori_loop(0, (NWIN - 2) // 2, loop_body, 0)
  window_body(NWIN - 2, 0)
  wc = wins[1]
  gstep(1, wc, 2, wc, setb, seta)
  gstep(2, wc, 3, wc, seta, setb)
  drain_scatters(seta)
  drain_gathers(setb)
  issue_scatters(3, wc, setb)
  drain_scatters(setb)

  plsc.subcore_barrier()

  pltpu.sync_copy(acc.at[pl.ds(s * SA, SA), :],
                  agg_o.at[c, pl.ds(s * SA, SA), :])


_agg = pl.kernel(
    _agg_body,
    out_type=jax.ShapeDtypeStruct((2, NPAD, HD), jnp.float32),
    mesh=_MESH,
    scratch_types=[
        pltpu.VMEM((W, CH), jnp.int32),
        pltpu.VMEM((W, CH), jnp.int32),
        pltpu.VMEM((W, CH), jnp.int32),
        pltpu.VMEM((W, CH), jnp.int32),
    ] + [pltpu.VMEM((CH, HD), jnp.float32) for _ in range(2 * K)] + [
        pltpu.VMEM_SHARED((NPAD, HD), jnp.float32),
        pltpu.VMEM_SHARED((NACC, HD), jnp.float32),
        pltpu.SemaphoreType.DMA,
        pltpu.SemaphoreType.DMA,
        pltpu.SemaphoreType.DMA,
    ],
    compiler_params=_SC_PARAMS,
)

CNCH = NCH // 2


def _cnt_body(dst, cnt_o, didx, zb, ones, cacc, ss):
  c = lax.axis_index("c")
  s = lax.axis_index("s")

  pltpu.async_copy(dst.at[s, pl.ds(c * CNCH, CNCH), :], didx, ss)

  z16 = jnp.zeros((16,), jnp.float32)
  one16 = jnp.full((16,), 1.0, jnp.float32)

  def fill_z(i, _):
    zb[i, :] = z16
    return 0

  def fill_o(i, _):
    ones[i, :] = one16
    return 0

  lax.fori_loop(0, 128, fill_z, 0)
  lax.fori_loop(0, CH, fill_o, 0)
  for k in range(SA // 128):
    pltpu.sync_copy(zb, cacc.at[pl.ds(s * SA + k * 128, 128), :])
  pltpu.make_async_copy(dst.at[s, pl.ds(c * CNCH, CNCH), :], didx, ss).wait()

  plsc.subcore_barrier()

  CK = 8

  def issue(m):
    for j in range(CK):
      pltpu.async_copy(ones, cacc.at[didx.at[CK * m + j]], ss, add=True)

  def drain():
    for j in range(CK):
      pltpu.make_async_copy(ones, cacc.at[didx.at[0]], ss).wait()

  issue(0)

  def loop_body(m, _):
    issue(m + 1)
    drain()
    return 0

  lax.fori_loop(0, CNCH // CK - 1, loop_body, 0)
  drain()

  plsc.subcore_barrier()

  pltpu.sync_copy(cacc.at[pl.ds(s * SA, SA), :],
                  cnt_o.at[c, pl.ds(s * SA, SA), :])


_cnt = pl.kernel(
    _cnt_body,
    out_type=jax.ShapeDtypeStruct((2, NPAD, 16), jnp.float32),
    mesh=_MESH,
    scratch_types=[
        pltpu.VMEM((CNCH, CH), jnp.int32),
        pltpu.VMEM((128, 16), jnp.float32),
        pltpu.VMEM((CH, 16), jnp.float32),
        pltpu.VMEM_SHARED((NACC, 16), jnp.float32),
        pltpu.SemaphoreType.DMA,
    ],
    compiler_params=_SC_PARAMS,
)


def _dense(p_ref, c_ref, h_ref, wl_ref, wr_ref, b_ref):
  cnt = c_ref[0, :, 0:1] + c_ref[1, :, 0:1]
  inv = 1.0 / jnp.maximum(cnt, 1.0)
  z = (jnp.dot(p_ref[0] * inv, wl_ref[0:HD, :],
               preferred_element_type=jnp.float32)
       + jnp.dot(p_ref[1] * inv, wl_ref[HD:D, :],
                 preferred_element_type=jnp.float32)
       + jnp.dot(h_ref[0], wr_ref[0:HD, :],
                 preferred_element_type=jnp.float32)
       + jnp.dot(h_ref[1], wr_ref[HD:D, :],
                 preferred_element_type=jnp.float32)
       + b_ref[...])
  return jnp.maximum(z, 0.0)


def _mid_body(p_ref, c_ref, h_ref, wl_ref, wr_ref, b_ref, o_ref):
  hn = _dense(p_ref, c_ref, h_ref, wl_ref, wr_ref, b_ref)
  o_ref[0] = hn[:, 0:HD]
  o_ref[1] = hn[:, HD:D]


def _fin_body(p_ref, c_ref, h_ref, wl_ref, wr_ref, b_ref, wo_ref, bo_ref,
              o_ref):
  hn = _dense(p_ref, c_ref, h_ref, wl_ref, wr_ref, b_ref)
  o_ref[...] = (jnp.dot(hn, wo_ref[...], preferred_element_type=jnp.float32)
                + bo_ref[...])


_P_SPEC = pl.BlockSpec((2, RB, HD), lambda i: (0, i, 0))
_C_SPEC = pl.BlockSpec((2, RB, 16), lambda i: (0, i, 0))
_W_SPEC = pl.BlockSpec((D, D), lambda i: (0, 0))
_B_SPEC = pl.BlockSpec((1, D), lambda i: (0, 0))

_mid = pl.pallas_call(
    _mid_body,
    grid=(NPAD // RB,),
    in_specs=[_P_SPEC, _C_SPEC, _P_SPEC, _W_SPEC, _W_SPEC, _B_SPEC],
    out_specs=_P_SPEC,
    out_shape=jax.ShapeDtypeStruct((2, NPAD, HD), jnp.float32),
)

_fin = pl.pallas_call(
    _fin_body,
    grid=(NPAD // RB,),
    in_specs=[_P_SPEC, _C_SPEC, _P_SPEC, _W_SPEC, _W_SPEC, _B_SPEC,
              _W_SPEC, _B_SPEC],
    out_specs=pl.BlockSpec((RB, D), lambda i: (i, 0)),
    out_shape=jax.ShapeDtypeStruct((NPAD, D), jnp.float32),
)


def kernel(x, edge_index, Wl1, Wr1, b1, Wl2, Wr2, b2, Wl3, Wr3, b3, Wo, bo):
  pad = jnp.full((EPAD - E,), N, jnp.int32)
  srcp = jnp.concatenate([edge_index[0], pad]).reshape(NT, NCH, CH)
  dstp = jnp.concatenate([edge_index[1], pad]).reshape(NT, NCH, CH)
  xp = jnp.pad(x, ((0, NPAD - N), (0, 0)))
  xs = jnp.stack([xp[:, 0:HD], xp[:, HD:D]])
  cnt = _cnt(dstp)
  srcp, dstp, xs2 = lax.optimization_barrier((srcp, dstp, (xs, cnt)))
  xs = xs2[0]
  agg1 = _agg(xs, srcp, dstp)
  h1 = _mid(agg1, cnt, xs, Wl1, Wr1, b1.reshape(1, D))
  agg2 = _agg(h1, srcp, dstp)
  h2 = _mid(agg2, cnt, h1, Wl2, Wr2, b2.reshape(1, D))
  agg3 = _agg(h2, srcp, dstp)
  outp = _fin(agg3, cnt, h2, Wl3, Wr3, b3.reshape(1, D), Wo, bo.reshape(1, D))
  return outp[:N]
